# baseline (speedup 1.0000x reference)
import jax
import jax.numpy as jnp
from jax.experimental import pallas as pl
from jax.experimental.pallas import tpu as pltpu

S = 512
BLOCK = 4096

_DN_T = (((1,), (1,)), ((), ()))


def _mmt(a, w):
    return jax.lax.dot_general(a, w, _DN_T, preferred_element_type=jnp.float32)


SPLIT = 8


def _net_kernel(x_ref, wv1_ref, wv2_ref, wq1_ref, bq1_ref, wq2_ref, bq2_ref,
                wvh1_ref, bvh1_ref, wvh2_ref, bvh2_ref, out_ref):
    sub = BLOCK // SPLIT
    for p in range(SPLIT):
        rows = pl.ds(p * sub, sub)
        s = x_ref[rows, :]
        h = jnp.maximum(_mmt(s, wv1_ref[...]), 0.0)
        v = _mmt(h, wv2_ref[...])
        hq = jnp.maximum(_mmt(v, wq1_ref[...]) + bq1_ref[...], 0.0)
        q = _mmt(hq, wq2_ref[...]) + bq2_ref[...]
        hv = jnp.maximum(_mmt(v, wvh1_ref[...]) + bvh1_ref[...], 0.0)
        vs = jnp.sum(hv * wvh2_ref[...], axis=1, keepdims=True) + bvh2_ref[...]
        out_ref[rows, :] = q - jnp.mean(q, axis=1, keepdims=True) + vs


def kernel(x, Wq1, bq1, Wq2, bq2, Wk1, Wk2, Wv1, Wv2,
           WQ1, bQ1, WQ2, bQ2, WV1, bV1, WV2, bV2):
    ba = x.shape[0]
    bq1_, bq2_, bvh1_, bvh2_ = bQ1, bQ2, bV1, bV2

    out_dim = WQ2.shape[0]
    grid = (ba // BLOCK,)

    def full(arr):
        return pl.BlockSpec(arr.shape, lambda i: (0,) * arr.ndim)

    return pl.pallas_call(
        _net_kernel,
        grid=grid,
        in_specs=[
            pl.BlockSpec((BLOCK, S), lambda i: (i, 0)),
            full(Wv1), full(Wv2), full(WQ1), full(bq1_), full(WQ2), full(bq2_),
            full(WV1), full(bvh1_), full(WV2), full(bvh2_),
        ],
        out_specs=pl.BlockSpec((BLOCK, out_dim), lambda i: (i, 0)),
        out_shape=jax.ShapeDtypeStruct((ba, out_dim), jnp.float32),
        compiler_params=pltpu.CompilerParams(
            dimension_semantics=("parallel",)),
    )(x, Wv1, Wv2, WQ1, bq1_, WQ2, bq2_, WV1, bvh1_, WV2, bvh2_)


# speedup vs baseline: 1.0437x; 1.0437x over previous
import jax
import jax.numpy as jnp
from jax.experimental import pallas as pl
from jax.experimental.pallas import tpu as pltpu

S = 512
BLOCK = 2048

_DN_T = (((1,), (1,)), ((), ()))


def _mmt(a, w):
    return jax.lax.dot_general(a, w, _DN_T, preferred_element_type=jnp.float32)


SPLIT = 2


def _net_kernel(x_ref, wv1_ref, wv2_ref, wq1_ref, bq1_ref, wq2_ref, bq2_ref,
                wvh1_ref, bvh1_ref, wvh2_ref, bvh2_ref, out_ref):
    sub = BLOCK // SPLIT
    for p in range(SPLIT):
        rows = pl.ds(p * sub, sub)
        s = x_ref[rows, :]
        h = jnp.maximum(_mmt(s, wv1_ref[...]), 0.0)
        v = _mmt(h, wv2_ref[...])
        hq = jnp.maximum(_mmt(v, wq1_ref[...]) + bq1_ref[...], 0.0)
        q = _mmt(hq, wq2_ref[...]) + bq2_ref[...]
        hv = jnp.maximum(_mmt(v, wvh1_ref[...]) + bvh1_ref[...], 0.0)
        vs = jnp.sum(hv * wvh2_ref[...], axis=1, keepdims=True) + bvh2_ref[...]
        out_ref[rows, :] = q - jnp.mean(q, axis=1, keepdims=True) + vs


def kernel(x, Wq1, bq1, Wq2, bq2, Wk1, Wk2, Wv1, Wv2,
           WQ1, bQ1, WQ2, bQ2, WV1, bV1, WV2, bV2):
    ba = x.shape[0]
    bq1_, bq2_, bvh1_, bvh2_ = bQ1, bQ2, bV1, bV2

    out_dim = WQ2.shape[0]
    grid = (ba // BLOCK,)

    def full(arr):
        return pl.BlockSpec(arr.shape, lambda i: (0,) * arr.ndim)

    return pl.pallas_call(
        _net_kernel,
        grid=grid,
        in_specs=[
            pl.BlockSpec((BLOCK, S), lambda i: (i, 0)),
            full(Wv1), full(Wv2), full(WQ1), full(bq1_), full(WQ2), full(bq2_),
            full(WV1), full(bvh1_), full(WV2), full(bvh2_),
        ],
        out_specs=pl.BlockSpec((BLOCK, out_dim), lambda i: (i, 0)),
        out_shape=jax.ShapeDtypeStruct((ba, out_dim), jnp.float32),
        compiler_params=pltpu.CompilerParams(
            dimension_semantics=("parallel",)),
    )(x, Wv1, Wv2, WQ1, bq1_, WQ2, bq2_, WV1, bvh1_, WV2, bvh2_)


# speedup vs baseline: 1.1412x; 1.0935x over previous
import jax
import jax.numpy as jnp
from jax.experimental import pallas as pl
from jax.experimental.pallas import tpu as pltpu

S = 512
BLOCK = 2048

_DN_T = (((1,), (1,)), ((), ()))


def _mmt(a, w):
    return jax.lax.dot_general(a, w, _DN_T, preferred_element_type=jnp.float32)


SPLIT = 4


def _net_kernel(x_ref, wv1_ref, wv2_ref, wq1_ref, bq1_ref, wq2_ref, bq2_ref,
                wvh1_ref, bvh1_ref, wvh2_ref, bvh2_ref, out_ref):
    _dn_fold = (((0,), (1,)), ((), ()))
    a_w = jax.lax.dot_general(wv2_ref[...], wq1_ref[...], _dn_fold,
                              preferred_element_type=jnp.float32)
    b_w = jax.lax.dot_general(wv2_ref[...], wvh1_ref[...], _dn_fold,
                              preferred_element_type=jnp.float32)
    _dn_n = (((1,), (0,)), ((), ()))
    sub = BLOCK // SPLIT
    for p in range(SPLIT):
        rows = pl.ds(p * sub, sub)
        s = x_ref[rows, :]
        h = jnp.maximum(_mmt(s, wv1_ref[...]), 0.0)
        hq = jnp.maximum(
            jax.lax.dot_general(h, a_w, _dn_n,
                                preferred_element_type=jnp.float32)
            + bq1_ref[...], 0.0)
        q = _mmt(hq, wq2_ref[...]) + bq2_ref[...]
        hv = jnp.maximum(
            jax.lax.dot_general(h, b_w, _dn_n,
                                preferred_element_type=jnp.float32)
            + bvh1_ref[...], 0.0)
        vs = jnp.sum(hv * wvh2_ref[...], axis=1, keepdims=True) + bvh2_ref[...]
        out_ref[rows, :] = q - jnp.mean(q, axis=1, keepdims=True) + vs


def kernel(x, Wq1, bq1, Wq2, bq2, Wk1, Wk2, Wv1, Wv2,
           WQ1, bQ1, WQ2, bQ2, WV1, bV1, WV2, bV2):
    ba = x.shape[0]
    bq1_, bq2_, bvh1_, bvh2_ = bQ1, bQ2, bV1, bV2

    out_dim = WQ2.shape[0]
    grid = (ba // BLOCK,)

    def full(arr):
        return pl.BlockSpec(arr.shape, lambda i: (0,) * arr.ndim)

    return pl.pallas_call(
        _net_kernel,
        grid=grid,
        in_specs=[
            pl.BlockSpec((BLOCK, S), lambda i: (i, 0)),
            full(Wv1), full(Wv2), full(WQ1), full(bq1_), full(WQ2), full(bq2_),
            full(WV1), full(bvh1_), full(WV2), full(bvh2_),
        ],
        out_specs=pl.BlockSpec((BLOCK, out_dim), lambda i: (i, 0)),
        out_shape=jax.ShapeDtypeStruct((ba, out_dim), jnp.float32),
        compiler_params=pltpu.CompilerParams(
            dimension_semantics=("parallel",)),
    )(x, Wv1, Wv2, WQ1, bq1_, WQ2, bq2_, WV1, bvh1_, WV2, bvh2_)
